# baseline (speedup 1.0000x reference)
import functools

import jax
import jax.numpy as jnp
from jax import lax
from jax.experimental import pallas as pl
from jax.experimental.pallas import tpu as pltpu
from jax.experimental.pallas import tpu_sc as plsc

N = 10000
E = 320000
D_IN = 128
D_EDGE = 16
D = 128
D_OUT = 16
L_LAYERS = 3
EPS = 1e-5

F32 = jnp.float32


_U32 = jnp.uint32
_DH = D // 2


def _pack_pair(a):
    lo = jax.lax.bitcast_convert_type(a[:, :_DH].astype(jnp.bfloat16), jnp.uint16)
    hi = jax.lax.bitcast_convert_type(a[:, _DH:].astype(jnp.bfloat16), jnp.uint16)
    return lo.astype(_U32) | (hi.astype(_U32) << 16)


def _wprep_body(we_ref, wedge_ref, bedge_ref, w16_ref, be_ref):
    for l in range(L_LAYERS):
        wl = we_ref[l]
        w16_ref[l] = jnp.dot(wedge_ref[...], wl, preferred_element_type=F32)
        be_ref[l] = jnp.dot(bedge_ref[...], wl, preferred_element_type=F32)


def _wprep(we, w_edge, b_edge):
    return pl.pallas_call(
        _wprep_body,
        out_shape=(
            jax.ShapeDtypeStruct((L_LAYERS, D_EDGE, D), F32),
            jax.ShapeDtypeStruct((L_LAYERS, 1, D), F32),
        ),
    )(we, w_edge, b_edge.reshape(1, D))


_EB = 8000


def _e_body(ea_ref, w16_ref, be_ref, o_ref):
    e = jnp.dot(ea_ref[...], w16_ref[...], preferred_element_type=F32) + be_ref[...]
    o_ref[...] = _pack_pair(e)


def _e_proj(edge_attr, w16_l, be_l):
    nblk = E // _EB
    return pl.pallas_call(
        _e_body,
        grid=(nblk,),
        in_specs=[
            pl.BlockSpec((_EB, D_EDGE), lambda i: (i, 0)),
            pl.BlockSpec((D_EDGE, D), lambda i: (0, 0)),
            pl.BlockSpec((1, D), lambda i: (0, 0)),
        ],
        out_specs=pl.BlockSpec((_EB, D // 2), lambda i: (i, 0)),
        out_shape=jax.ShapeDtypeStruct((E, D // 2), _U32),
    )(edge_attr, w16_l, be_l)


def _proj_kqvs(h, wk_ref, wq_ref, wv_ref, ws_ref, bk_ref, bq_ref, bv_ref,
               bs_ref, k_ref, qv_ref, s_ref):
    q = jnp.dot(h, wq_ref[...], preferred_element_type=F32) + bq_ref[...]
    v = jnp.dot(h, wv_ref[...], preferred_element_type=F32) + bv_ref[...]
    k_ref[...] = jnp.dot(h, wk_ref[...], preferred_element_type=F32) + bk_ref[...]
    qv_ref[:, :_DH] = _pack_pair(q)
    qv_ref[:, _DH:] = _pack_pair(v)
    s_ref[...] = jnp.dot(h, ws_ref[...], preferred_element_type=F32) + bs_ref[...]


_KQVS_OUT = (
    jax.ShapeDtypeStruct((N, D), F32),
    jax.ShapeDtypeStruct((N, D), _U32),
    jax.ShapeDtypeStruct((N, D), F32),
)


def _first_body(x_ref, wn_ref, bn_ref, wk_ref, wq_ref, wv_ref, ws_ref,
                bk_ref, bq_ref, bv_ref, bs_ref, h_ref, k_ref, qv_ref, s_ref):
    h = jnp.dot(x_ref[...], wn_ref[...], preferred_element_type=F32) + bn_ref[...]
    h_ref[...] = h
    _proj_kqvs(h, wk_ref, wq_ref, wv_ref, ws_ref, bk_ref, bq_ref, bv_ref,
               bs_ref, k_ref, qv_ref, s_ref)


def _first(x, w_node, b_node, wk_l, wq_l, wv_l, ws_l, bk_l, bq_l, bv_l, bs_l):
    return pl.pallas_call(
        _first_body,
        out_shape=(jax.ShapeDtypeStruct((N, D), F32),) + _KQVS_OUT,
    )(x, w_node, b_node.reshape(1, D), wk_l, wq_l, wv_l, ws_l,
      bk_l.reshape(1, D), bq_l.reshape(1, D), bv_l.reshape(1, D),
      bs_l.reshape(1, D))


def _bn_update(aggr_ref, s_ref, h_ref, g_ref, b_ref):
    n_new = aggr_ref[0, :N, :] + aggr_ref[1, :N, :] + s_ref[...]
    mean = jnp.mean(n_new, axis=0, keepdims=True)
    cent = n_new - mean
    var = jnp.mean(cent * cent, axis=0, keepdims=True)
    bn = g_ref[...] * cent * lax.rsqrt(var + EPS) + b_ref[...]
    return (h_ref[...] + jnp.maximum(bn, 0.0)) * 0.5


def _mid_body(aggr_ref, s_ref, h_ref, g_ref, b_ref, wk_ref, wq_ref, wv_ref,
              ws_ref, bk_ref, bq_ref, bv_ref, bs_ref,
              h_out, k_ref, qv_ref, s_out):
    h_new = _bn_update(aggr_ref, s_ref, h_ref, g_ref, b_ref)
    h_out[...] = h_new
    _proj_kqvs(h_new, wk_ref, wq_ref, wv_ref, ws_ref, bk_ref, bq_ref, bv_ref,
               bs_ref, k_ref, qv_ref, s_out)


def _mid(aggr2, s, h, gamma_l, beta_l, wk_l, wq_l, wv_l, ws_l,
         bk_l, bq_l, bv_l, bs_l):
    return pl.pallas_call(
        _mid_body,
        out_shape=(jax.ShapeDtypeStruct((N, D), F32),) + _KQVS_OUT,
    )(aggr2, s, h, gamma_l.reshape(1, D), beta_l.reshape(1, D),
      wk_l, wq_l, wv_l, ws_l, bk_l.reshape(1, D), bq_l.reshape(1, D),
      bv_l.reshape(1, D), bs_l.reshape(1, D))


def _last_body(aggr_ref, s_ref, h_ref, g_ref, b_ref, wh_ref, bh_ref, o_ref):
    h_new = _bn_update(aggr_ref, s_ref, h_ref, g_ref, b_ref)
    o_ref[...] = (
        jnp.dot(h_new, wh_ref[...], preferred_element_type=F32) + bh_ref[...]
    )


def _last(aggr2, s, h, gamma_l, beta_l, w_head, b_head):
    return pl.pallas_call(
        _last_body,
        out_shape=jax.ShapeDtypeStruct((N, D_OUT), F32),
    )(aggr2, s, h, gamma_l.reshape(1, D), beta_l.reshape(1, D),
      w_head, b_head.reshape(1, D_OUT))


_NT = 16
_NC = 2
_CH = 40
_EPT = E // (_NT * _NC)
_NCHUNK = _EPT // _CH
_NPAIR = _NCHUNK // 2
_NPAD = 10240
_RPT = _NPAD // _NT


def _sc_edge_body(kqv_hbm, e_hbm, cix_hbm, dst_hbm, z_hbm, out_hbm,
                  is0, is1, is2, is3, id0, id1, id2, id3,
                  kqb0, kqb1, eb0, eb1, gb0, gb1,
                  shared, gsem, ssem, isem, hsem):
    cid = lax.axis_index("c")
    sid = lax.axis_index("s")
    wid = sid * _NC + cid
    base0 = wid * _EPT

    pltpu.async_copy(
        z_hbm.at[pl.ds(sid * _RPT, _RPT)],
        shared.at[pl.ds(sid * _RPT, _RPT)],
        hsem,
    ).wait()
    plsc.subcore_barrier()

    isb = (is0, is1, is2, is3)
    idb = (id0, id1, id2, id3)
    kqbufs = (kqb0, kqb1)
    ebufs = (eb0, eb1)
    gbufs = (gb0, gb1)

    def start_idx(ci, r):
        gci = wid * _NCHUNK + ci
        pltpu.async_copy(cix_hbm.at[pl.ds(gci * 2 * _CH, 2 * _CH)], isb[r], isem)
        pltpu.async_copy(dst_hbm.at[pl.ds(base0 + ci * _CH, _CH)], idb[r], isem)

    def wait_idx(r):
        pltpu.make_async_copy(cix_hbm.at[pl.ds(0, 2 * _CH)], isb[r], isem).wait()
        pltpu.make_async_copy(dst_hbm.at[pl.ds(0, _CH)], idb[r], isem).wait()

    def start_gathers(ci, r, b):
        pltpu.async_copy(e_hbm.at[pl.ds(base0 + ci * _CH, _CH)], ebufs[b], gsem)
        pltpu.async_copy(kqv_hbm.at[isb[r]], kqbufs[b], gsem)

    def wait_gathers(r, b):
        pltpu.make_async_copy(e_hbm.at[pl.ds(0, _CH)], ebufs[b], gsem).wait()
        pltpu.make_async_copy(kqv_hbm.at[isb[r]], kqbufs[b], gsem).wait()

    hi_mask = jnp.uint32(0xFFFF0000)

    def _lo(w):
        return jax.lax.bitcast_convert_type(w << 16, F32)

    def _hi(w):
        return jax.lax.bitcast_convert_type(w & hi_mask, F32)

    def compute(b):
        kqb, eb, gb = kqbufs[b], ebufs[b], gbufs[b]

        def edge_body(i, c2):
            for g in range(_DH // 16):
                sl = pl.ds(g * 16, 16)
                sh = pl.ds(_DH + g * 16, 16)
                we_ = eb[i, sl]
                wq = kqb[_CH + i, sl]
                wv = kqb[_CH + i, sh]
                k_lo = jax.lax.bitcast_convert_type(kqb[i, sl], F32)
                k_hi = jax.lax.bitcast_convert_type(kqb[i, sh], F32)
                gate_lo = jnp.maximum(k_lo + _lo(we_) + _lo(wq), 0.0)
                gate_hi = jnp.maximum(k_hi + _hi(we_) + _hi(wq), 0.0)
                gb[i, sl] = gate_lo * _lo(wv)
                gb[i, sh] = gate_hi * _hi(wv)
            return c2

        lax.fori_loop(0, _CH, edge_body, 0, unroll=2)

    def start_scatter(r, b):
        pltpu.async_copy(gbufs[b], shared.at[idb[r]], ssem, add=True)

    def wait_scatter(r, b):
        pltpu.make_async_copy(gbufs[b], shared.at[idb[r]], ssem).wait()

    def half_step(c, j, r):
        b = r % 2
        wait_gathers(r, b)

        @pl.when(j + c > 0)
        def _():
            wait_scatter((r - 1) % 4, 1 - b)

        wait_idx((r + 1) % 4)
        start_gathers(c + 1, (r + 1) % 4, 1 - b)
        start_idx(c + 2, (r + 2) % 4)
        compute(b)
        start_scatter(r, b)

    start_idx(0, 0)
    wait_idx(0)
    start_gathers(0, 0, 0)
    start_idx(1, 1)

    def quad_body(j, carry):
        c0 = 4 * j
        for u in range(4):
            half_step(c0 + u, j, u)
        return carry

    lax.fori_loop(0, (_NCHUNK - 2) // 4, quad_body, 0)
    wait_gathers(0, 0)
    wait_scatter(3, 1)
    wait_idx(1)
    start_gathers(_NCHUNK - 1, 1, 1)
    compute(0)
    start_scatter(0, 0)
    wait_gathers(1, 1)
    wait_scatter(0, 0)
    compute(1)
    start_scatter(1, 1)
    wait_scatter(1, 1)

    plsc.subcore_barrier()
    pltpu.async_copy(
        shared.at[pl.ds(sid * _RPT, _RPT)],
        out_hbm.at[cid, pl.ds(sid * _RPT, _RPT)],
        hsem,
    ).wait()


def _sc_edge(kqv_tab, e, cix, dst, zeros):
    mesh = plsc.VectorSubcoreMesh(core_axis_name="c", subcore_axis_name="s")
    cix_t = pltpu.VMEM((2 * _CH,), jnp.int32)
    idx_t = pltpu.VMEM((_CH,), jnp.int32)
    fn = pl.kernel(
        _sc_edge_body,
        out_type=jax.ShapeDtypeStruct((_NC, _NPAD, D), F32),
        mesh=mesh,
        compiler_params=pltpu.CompilerParams(needs_layout_passes=False),
        scratch_types=[
            cix_t, cix_t, cix_t, cix_t,
            idx_t, idx_t, idx_t, idx_t,
            pltpu.VMEM((2 * _CH, D), _U32),
            pltpu.VMEM((2 * _CH, D), _U32),
            pltpu.VMEM((_CH, _DH), _U32),
            pltpu.VMEM((_CH, _DH), _U32),
            pltpu.VMEM((_CH, D), F32),
            pltpu.VMEM((_CH, D), F32),
            pltpu.VMEM_SHARED((_NPAD, D), F32),
            pltpu.SemaphoreType.DMA,
            pltpu.SemaphoreType.DMA,
            pltpu.SemaphoreType.DMA,
            pltpu.SemaphoreType.DMA,
        ],
    )
    return fn(kqv_tab, e, cix, dst, zeros)


def kernel(x, edge_index, edge_attr, w_node, b_node, w_edge, b_edge,
           wk, bk, wq, bq, wv, bv, ws, bs, we, gamma, beta, w_head, b_head):
    src = edge_index[0]
    dst = edge_index[1]
    cix = jnp.concatenate(
        [dst.reshape(-1, _CH), src.reshape(-1, _CH) + N], axis=1
    ).reshape(-1)
    w16, be = _wprep(we, w_edge, b_edge)
    zeros = jnp.zeros((_NPAD, D), F32)
    e = _e_proj(edge_attr, w16[0], be[0])
    h, k, qv, s = _first(x, w_node, b_node, wk[0], wq[0], wv[0], ws[0],
                         bk[0], bq[0], bv[0], bs[0])
    for l in range(L_LAYERS):
        kqv_tab = jnp.concatenate(
            [jax.lax.bitcast_convert_type(k, _U32), qv], axis=0)
        aggr2 = _sc_edge(kqv_tab, e, cix, dst, zeros)
        if l + 1 < L_LAYERS:
            e = _e_proj(edge_attr, w16[l + 1], be[l + 1])
        if l + 1 < L_LAYERS:
            h, k, qv, s = _mid(aggr2, s, h, gamma[l], beta[l],
                               wk[l + 1], wq[l + 1], wv[l + 1], ws[l + 1],
                               bk[l + 1], bq[l + 1], bv[l + 1], bs[l + 1])
        else:
            return _last(aggr2, s, h, gamma[l], beta[l], w_head, b_head)


# speedup vs baseline: 1.0546x; 1.0546x over previous
import functools

import jax
import jax.numpy as jnp
from jax import lax
from jax.experimental import pallas as pl
from jax.experimental.pallas import tpu as pltpu
from jax.experimental.pallas import tpu_sc as plsc

N = 10000
E = 320000
D_IN = 128
D_EDGE = 16
D = 128
D_OUT = 16
L_LAYERS = 3
EPS = 1e-5

F32 = jnp.float32


_U32 = jnp.uint32
_DH = D // 2


def _pack_pair(a):
    lo = jax.lax.bitcast_convert_type(a[:, :_DH].astype(jnp.bfloat16), jnp.uint16)
    hi = jax.lax.bitcast_convert_type(a[:, _DH:].astype(jnp.bfloat16), jnp.uint16)
    return lo.astype(_U32) | (hi.astype(_U32) << 16)


def _wprep_body(we_ref, wedge_ref, bedge_ref, w16_ref, be_ref):
    for l in range(L_LAYERS):
        wl = we_ref[l]
        w16_ref[l] = jnp.dot(wedge_ref[...], wl, preferred_element_type=F32)
        be_ref[l] = jnp.dot(bedge_ref[...], wl, preferred_element_type=F32)


def _wprep(we, w_edge, b_edge):
    return pl.pallas_call(
        _wprep_body,
        out_shape=(
            jax.ShapeDtypeStruct((L_LAYERS, D_EDGE, D), F32),
            jax.ShapeDtypeStruct((L_LAYERS, 1, D), F32),
        ),
    )(we, w_edge, b_edge.reshape(1, D))


_EB = 8000


def _e_body(ea_ref, w16_ref, be_ref, o_ref):
    e = jnp.dot(ea_ref[...], w16_ref[...], preferred_element_type=F32) + be_ref[...]
    o_ref[...] = _pack_pair(e)


def _e_proj(edge_attr, w16_l, be_l):
    nblk = E // _EB
    return pl.pallas_call(
        _e_body,
        grid=(nblk,),
        in_specs=[
            pl.BlockSpec((_EB, D_EDGE), lambda i: (i, 0)),
            pl.BlockSpec((D_EDGE, D), lambda i: (0, 0)),
            pl.BlockSpec((1, D), lambda i: (0, 0)),
        ],
        out_specs=pl.BlockSpec((_EB, D // 2), lambda i: (i, 0)),
        out_shape=jax.ShapeDtypeStruct((E, D // 2), _U32),
    )(edge_attr, w16_l, be_l)


def _proj_kqvs(h, wk_ref, wq_ref, wv_ref, ws_ref, bk_ref, bq_ref, bv_ref,
               bs_ref, k_ref, qv_ref, s_ref):
    q = jnp.dot(h, wq_ref[...], preferred_element_type=F32) + bq_ref[...]
    v = jnp.dot(h, wv_ref[...], preferred_element_type=F32) + bv_ref[...]
    k_ref[...] = jnp.dot(h, wk_ref[...], preferred_element_type=F32) + bk_ref[...]
    qv_ref[:, :_DH] = _pack_pair(q)
    qv_ref[:, _DH:] = _pack_pair(v)
    s_ref[...] = jnp.dot(h, ws_ref[...], preferred_element_type=F32) + bs_ref[...]


_KQVS_OUT = (
    jax.ShapeDtypeStruct((N, D), F32),
    jax.ShapeDtypeStruct((N, D), _U32),
    jax.ShapeDtypeStruct((N, D), F32),
)


def _first_body(x_ref, wn_ref, bn_ref, wk_ref, wq_ref, wv_ref, ws_ref,
                bk_ref, bq_ref, bv_ref, bs_ref, h_ref, k_ref, qv_ref, s_ref):
    h = jnp.dot(x_ref[...], wn_ref[...], preferred_element_type=F32) + bn_ref[...]
    h_ref[...] = h
    _proj_kqvs(h, wk_ref, wq_ref, wv_ref, ws_ref, bk_ref, bq_ref, bv_ref,
               bs_ref, k_ref, qv_ref, s_ref)


def _first(x, w_node, b_node, wk_l, wq_l, wv_l, ws_l, bk_l, bq_l, bv_l, bs_l):
    return pl.pallas_call(
        _first_body,
        out_shape=(jax.ShapeDtypeStruct((N, D), F32),) + _KQVS_OUT,
    )(x, w_node, b_node.reshape(1, D), wk_l, wq_l, wv_l, ws_l,
      bk_l.reshape(1, D), bq_l.reshape(1, D), bv_l.reshape(1, D),
      bs_l.reshape(1, D))


def _bn_update(aggr_ref, s_ref, h_ref, g_ref, b_ref):
    n_new = aggr_ref[0, :N, :] + aggr_ref[1, :N, :] + s_ref[...]
    mean = jnp.mean(n_new, axis=0, keepdims=True)
    cent = n_new - mean
    var = jnp.mean(cent * cent, axis=0, keepdims=True)
    bn = g_ref[...] * cent * lax.rsqrt(var + EPS) + b_ref[...]
    return (h_ref[...] + jnp.maximum(bn, 0.0)) * 0.5


def _mid_body(aggr_ref, s_ref, h_ref, g_ref, b_ref, wk_ref, wq_ref, wv_ref,
              ws_ref, bk_ref, bq_ref, bv_ref, bs_ref,
              h_out, k_ref, qv_ref, s_out):
    h_new = _bn_update(aggr_ref, s_ref, h_ref, g_ref, b_ref)
    h_out[...] = h_new
    _proj_kqvs(h_new, wk_ref, wq_ref, wv_ref, ws_ref, bk_ref, bq_ref, bv_ref,
               bs_ref, k_ref, qv_ref, s_out)


def _mid(aggr2, s, h, gamma_l, beta_l, wk_l, wq_l, wv_l, ws_l,
         bk_l, bq_l, bv_l, bs_l):
    return pl.pallas_call(
        _mid_body,
        out_shape=(jax.ShapeDtypeStruct((N, D), F32),) + _KQVS_OUT,
    )(aggr2, s, h, gamma_l.reshape(1, D), beta_l.reshape(1, D),
      wk_l, wq_l, wv_l, ws_l, bk_l.reshape(1, D), bq_l.reshape(1, D),
      bv_l.reshape(1, D), bs_l.reshape(1, D))


def _last_body(aggr_ref, s_ref, h_ref, g_ref, b_ref, wh_ref, bh_ref, o_ref):
    h_new = _bn_update(aggr_ref, s_ref, h_ref, g_ref, b_ref)
    o_ref[...] = (
        jnp.dot(h_new, wh_ref[...], preferred_element_type=F32) + bh_ref[...]
    )


def _last(aggr2, s, h, gamma_l, beta_l, w_head, b_head):
    return pl.pallas_call(
        _last_body,
        out_shape=jax.ShapeDtypeStruct((N, D_OUT), F32),
    )(aggr2, s, h, gamma_l.reshape(1, D), beta_l.reshape(1, D),
      w_head, b_head.reshape(1, D_OUT))


_NT = 16
_NC = 2
_CH = 40
_EPT = E // (_NT * _NC)
_NCHUNK = _EPT // _CH
_NPAIR = _NCHUNK // 2
_NPAD = 10240
_RPT = _NPAD // _NT


def _sc_edge_body(k_hbm, qv_hbm, e_hbm, src_hbm, dst_hbm, z_hbm, out_hbm,
                  is0, is1, is2, is3, id0, id1, id2, id3,
                  kb0, kb1, qvb0, qvb1, eb0, eb1, gb0, gb1,
                  shared, gsem, ssem, isem, hsem):
    cid = lax.axis_index("c")
    sid = lax.axis_index("s")
    wid = sid * _NC + cid
    base0 = wid * _EPT

    zcopy = pltpu.async_copy(
        z_hbm.at[pl.ds(sid * _RPT, _RPT)],
        shared.at[pl.ds(sid * _RPT, _RPT)],
        hsem,
    )

    isb = (is0, is1, is2, is3)
    idb = (id0, id1, id2, id3)
    kbufs = (kb0, kb1)
    qvbufs = (qvb0, qvb1)
    ebufs = (eb0, eb1)
    gbufs = (gb0, gb1)

    def start_idx(ci, r):
        pltpu.async_copy(src_hbm.at[pl.ds(base0 + ci * _CH, _CH)], isb[r], isem)
        pltpu.async_copy(dst_hbm.at[pl.ds(base0 + ci * _CH, _CH)], idb[r], isem)

    def wait_idx(r):
        pltpu.make_async_copy(src_hbm.at[pl.ds(0, _CH)], isb[r], isem).wait()
        pltpu.make_async_copy(dst_hbm.at[pl.ds(0, _CH)], idb[r], isem).wait()

    def start_gathers(ci, r, b):
        pltpu.async_copy(e_hbm.at[pl.ds(base0 + ci * _CH, _CH)], ebufs[b], gsem)
        pltpu.async_copy(k_hbm.at[idb[r]], kbufs[b], gsem)
        pltpu.async_copy(qv_hbm.at[isb[r]], qvbufs[b], gsem)

    def wait_gathers(r, b):
        pltpu.make_async_copy(e_hbm.at[pl.ds(0, _CH)], ebufs[b], gsem).wait()
        pltpu.make_async_copy(k_hbm.at[idb[r]], kbufs[b], gsem).wait()
        pltpu.make_async_copy(qv_hbm.at[isb[r]], qvbufs[b], gsem).wait()

    hi_mask = jnp.uint32(0xFFFF0000)

    def _lo(w):
        return jax.lax.bitcast_convert_type(w << 16, F32)

    def _hi(w):
        return jax.lax.bitcast_convert_type(w & hi_mask, F32)

    def compute(b):
        kb, qvb, eb, gb = kbufs[b], qvbufs[b], ebufs[b], gbufs[b]

        def edge_body(i, c2):
            for g in range(_DH // 16):
                sl = pl.ds(g * 16, 16)
                sh = pl.ds(_DH + g * 16, 16)
                we_ = eb[i, sl]
                wq = qvb[i, sl]
                wv = qvb[i, sh]
                gate_lo = jnp.maximum(kb[i, sl] + _lo(we_) + _lo(wq), 0.0)
                gate_hi = jnp.maximum(kb[i, sh] + _hi(we_) + _hi(wq), 0.0)
                gb[i, sl] = gate_lo * _lo(wv)
                gb[i, sh] = gate_hi * _hi(wv)
            return c2

        lax.fori_loop(0, _CH, edge_body, 0, unroll=4)

    def start_scatter(r, b):
        pltpu.async_copy(gbufs[b], shared.at[idb[r]], ssem, add=True)

    def wait_scatter(r, b):
        pltpu.make_async_copy(gbufs[b], shared.at[idb[r]], ssem).wait()

    def half_step(c, j, r):
        b = r % 2
        wait_gathers(r, b)

        @pl.when(j + c > 0)
        def _():
            wait_scatter((r - 1) % 4, 1 - b)

        wait_idx((r + 1) % 4)
        start_gathers(c + 1, (r + 1) % 4, 1 - b)
        start_idx(c + 2, (r + 2) % 4)
        compute(b)
        start_scatter(r, b)

    start_idx(0, 0)
    wait_idx(0)
    start_gathers(0, 0, 0)
    start_idx(1, 1)
    zcopy.wait()
    plsc.subcore_barrier()

    def quad_body(j, carry):
        c0 = 4 * j
        for u in range(4):
            half_step(c0 + u, j, u)
        return carry

    lax.fori_loop(0, (_NCHUNK - 2) // 4, quad_body, 0)
    wait_gathers(0, 0)
    wait_scatter(3, 1)
    wait_idx(1)
    start_gathers(_NCHUNK - 1, 1, 1)
    compute(0)
    start_scatter(0, 0)
    wait_gathers(1, 1)
    wait_scatter(0, 0)
    compute(1)
    start_scatter(1, 1)
    wait_scatter(1, 1)

    plsc.subcore_barrier()
    pltpu.async_copy(
        shared.at[pl.ds(sid * _RPT, _RPT)],
        out_hbm.at[cid, pl.ds(sid * _RPT, _RPT)],
        hsem,
    ).wait()


def _sc_edge(k, qv, e, src, dst, zeros):
    mesh = plsc.VectorSubcoreMesh(core_axis_name="c", subcore_axis_name="s")
    idx_t = pltpu.VMEM((_CH,), jnp.int32)
    fn = pl.kernel(
        _sc_edge_body,
        out_type=jax.ShapeDtypeStruct((_NC, _NPAD, D), F32),
        mesh=mesh,
        compiler_params=pltpu.CompilerParams(needs_layout_passes=False),
        scratch_types=[
            idx_t, idx_t, idx_t, idx_t,
            idx_t, idx_t, idx_t, idx_t,
            pltpu.VMEM((_CH, D), F32),
            pltpu.VMEM((_CH, D), F32),
            pltpu.VMEM((_CH, D), _U32),
            pltpu.VMEM((_CH, D), _U32),
            pltpu.VMEM((_CH, _DH), _U32),
            pltpu.VMEM((_CH, _DH), _U32),
            pltpu.VMEM((_CH, D), F32),
            pltpu.VMEM((_CH, D), F32),
            pltpu.VMEM_SHARED((_NPAD, D), F32),
            pltpu.SemaphoreType.DMA,
            pltpu.SemaphoreType.DMA,
            pltpu.SemaphoreType.DMA,
            pltpu.SemaphoreType.DMA,
        ],
    )
    return fn(k, qv, e, src, dst, zeros)


def kernel(x, edge_index, edge_attr, w_node, b_node, w_edge, b_edge,
           wk, bk, wq, bq, wv, bv, ws, bs, we, gamma, beta, w_head, b_head):
    src = edge_index[0]
    dst = edge_index[1]
    w16, be = _wprep(we, w_edge, b_edge)
    zeros = jnp.zeros((_NPAD, D), F32)
    e = _e_proj(edge_attr, w16[0], be[0])
    h, k, qv, s = _first(x, w_node, b_node, wk[0], wq[0], wv[0], ws[0],
                         bk[0], bq[0], bv[0], bs[0])
    for l in range(L_LAYERS):
        aggr2 = _sc_edge(k, qv, e, src, dst, zeros)
        if l + 1 < L_LAYERS:
            e = _e_proj(edge_attr, w16[l + 1], be[l + 1])
        if l + 1 < L_LAYERS:
            h, k, qv, s = _mid(aggr2, s, h, gamma[l], beta[l],
                               wk[l + 1], wq[l + 1], wv[l + 1], ws[l + 1],
                               bk[l + 1], bq[l + 1], bv[l + 1], bs[l + 1])
        else:
            return _last(aggr2, s, h, gamma[l], beta[l], w_head, b_head)
